# baseline (speedup 1.0000x reference)
import functools

import jax
import jax.numpy as jnp
from jax.experimental import pallas as pl
from jax.experimental.pallas import tpu as pltpu
from jax.experimental.pallas import tpu_sc as plsc

_VOCAB = 100000
_DIM = 64
_BATCH = 1024

_TV = 2048
_NBUF = 4
_WBUF = 8
_N_TILES = _VOCAB // _TV

_SC_CORES = 2
_B_PER_CORE = _BATCH // _SC_CORES


def _gather_rows(embed_table, input_ids):
    @functools.partial(
        pl.kernel,
        out_type=jax.ShapeDtypeStruct((_BATCH, _DIM), embed_table.dtype),
        mesh=plsc.ScalarSubcoreMesh(axis_name="core", num_cores=_SC_CORES),
        scratch_types=[
            pltpu.SMEM((_BATCH,), jnp.int32),
            pltpu.SemaphoreType.DMA,
            pltpu.SemaphoreType.DMA,
        ],
    )
    def sc_gather(table_hbm, idx_hbm, out_hbm, idx_s, isem, sem):
        core = jax.lax.axis_index("core")
        base = core * _B_PER_CORE
        pltpu.async_copy(idx_hbm, idx_s, isem).wait()

        @pl.loop(0, _B_PER_CORE)
        def _(j):
            pltpu.make_async_copy(
                table_hbm.at[idx_s[base + j]],
                out_hbm.at[base + j],
                sem,
            ).start()

        pltpu.make_async_copy(
            table_hbm.at[pl.ds(0, _B_PER_CORE)],
            out_hbm.at[pl.ds(base, _B_PER_CORE)],
            sem,
        ).wait()

    return sc_gather(embed_table, input_ids)


def _proj_body(h_ref, b_ref, w_hbm, o_hbm, obuf, wbuf, osems, wsems):
    i = pl.program_id(0)
    slot = jax.lax.rem(i, _NBUF)
    wslot = jax.lax.rem(i, _WBUF)

    @pl.when(i == 0)
    def _():
        for k in range(_WBUF):
            pltpu.make_async_copy(
                w_hbm.at[pl.ds(k * _TV, _TV), :],
                wbuf.at[k],
                wsems.at[k],
            ).start()

    pltpu.make_async_copy(
        w_hbm.at[pl.ds(i * _TV, _TV), :],
        wbuf.at[wslot],
        wsems.at[wslot],
    ).wait()

    @pl.when(i >= _NBUF)
    def _():
        pltpu.make_async_copy(
            obuf.at[slot],
            o_hbm.at[:, pl.ds((i - _NBUF) * _TV, _TV)],
            osems.at[slot],
        ).wait()

    hb = h_ref[...].astype(jnp.bfloat16)
    wb = wbuf[wslot].astype(jnp.bfloat16)
    acc = jax.lax.dot_general(
        hb, wb, (((1,), (1,)), ((), ())),
        preferred_element_type=jnp.float32,
    )
    obuf[slot] = acc + b_ref[...][None, :]

    pltpu.make_async_copy(
        obuf.at[slot],
        o_hbm.at[:, pl.ds(i * _TV, _TV)],
        osems.at[slot],
    ).start()

    @pl.when(i + _WBUF < _N_TILES)
    def _():
        pltpu.make_async_copy(
            w_hbm.at[pl.ds((i + _WBUF) * _TV, _TV), :],
            wbuf.at[wslot],
            wsems.at[wslot],
        ).start()

    @pl.when(i == _N_TILES - 1)
    def _():
        for k in range(_NBUF):
            step = _N_TILES - _NBUF + k
            s = step % _NBUF
            pltpu.make_async_copy(
                obuf.at[s],
                o_hbm.at[:, pl.ds(step * _TV, _TV)],
                osems.at[s],
            ).wait()


def _tail_body(h_ref, w_ref, b_ref, _, o_ref):
    hb = h_ref[...].astype(jnp.bfloat16)
    wb = w_ref[...].astype(jnp.bfloat16)
    acc = jax.lax.dot_general(
        hb, wb, (((1,), (1,)), ((), ())),
        preferred_element_type=jnp.float32,
    )
    o_ref[...] = acc + b_ref[...][None, :]


@functools.partial(jax.jit, static_argnames=("interpret",))
def _project(h, lin_w, lin_b, interpret=False):
    main = pl.pallas_call(
        _proj_body,
        grid=(_N_TILES,),
        in_specs=[
            pl.BlockSpec((_BATCH, _DIM), lambda i: (0, 0)),
            pl.BlockSpec((_TV,), lambda i: (i,)),
            pl.BlockSpec(memory_space=pl.ANY),
        ],
        out_specs=pl.BlockSpec(memory_space=pl.ANY),
        out_shape=jax.ShapeDtypeStruct((_BATCH, _VOCAB), jnp.float32),
        scratch_shapes=[
            pltpu.VMEM((_NBUF, _BATCH, _TV), jnp.float32),
            pltpu.VMEM((_WBUF, _TV, _DIM), jnp.float32),
            pltpu.SemaphoreType.DMA((_NBUF,)),
            pltpu.SemaphoreType.DMA((_WBUF,)),
        ],
        compiler_params=pltpu.CompilerParams(
            dimension_semantics=("arbitrary",),
        ),
        interpret=interpret,
    )(h, lin_b, lin_w)

    return pl.pallas_call(
        _tail_body,
        grid=(1,),
        in_specs=[
            pl.BlockSpec((_BATCH, _DIM), lambda i: (0, 0)),
            pl.BlockSpec((_TV, _DIM), lambda i: (_N_TILES, 0)),
            pl.BlockSpec((_TV,), lambda i: (_N_TILES,)),
            pl.BlockSpec(memory_space=pl.ANY),
        ],
        out_specs=pl.BlockSpec((_BATCH, _TV), lambda i: (0, _N_TILES)),
        out_shape=jax.ShapeDtypeStruct((_BATCH, _VOCAB), jnp.float32),
        input_output_aliases={3: 0},
        interpret=interpret,
    )(h, lin_w, lin_b, main)


def kernel(input_ids, embed_table, lin_w, lin_b):
    h = _gather_rows(embed_table, input_ids)
    return _project(h, lin_w, lin_b)
